# baseline (speedup 1.0000x reference)
import functools

import jax
import jax.numpy as jnp
import numpy as np
from jax.experimental import pallas as pl
from jax.experimental.pallas import tpu as pltpu

_B = 64
_D = 192
_SCALE_NS = (16, 64, 256)
_K = 4


@functools.lru_cache(maxsize=1)
def _rng_consts():
    with jax.ensure_compile_time_eval():
        return _rng_consts_impl()


def _rng_consts_impl():
    base = jax.random.key(42)
    idx0_cols = []
    gumbels = []
    for si, n in enumerate(_SCALE_NS):
        keys = jax.random.split(jax.random.fold_in(base, si), _B)

        def per_key(kk, n=n):
            key, sub = jax.random.split(kk)
            i0 = jax.random.randint(sub, (), 0, n)
            gs = []
            for _ in range(_K - 1):
                key, sub = jax.random.split(key)
                gs.append(jax.random.gumbel(sub, (n,), jnp.float32))
            return i0, jnp.stack(gs, axis=1)

        i0, g = jax.vmap(per_key)(keys)
        idx0_cols.append(np.asarray(i0, np.int32))
        gumbels.append(np.asarray(g, np.float32))
    idx0s = np.stack(idx0_cols, axis=1)
    return idx0s, gumbels


def _pool_matrix(g_out, g_in):
    m = np.zeros((g_out * g_out, g_in * g_in), np.float32)
    for n in range(g_in * g_in):
        a, bcol = n // g_in, n % g_in
        m[(a // 2) * g_out + (bcol // 2), n] = 0.25
    return m


def _kmeans_pp_steps(feats_ref, n, eg_ref, idx0, out_ref, out_base):
    inf = jnp.float32(np.inf)
    min_d = jnp.full((n, 1), inf, jnp.float32)
    iota = jax.lax.broadcasted_iota(jnp.int32, (n, 1), 0)
    idx = idx0
    for t in range(_K):
        center = feats_ref[pl.ds(idx, 1), :]
        out_ref[0, pl.ds(out_base + t, 1), :] = center
        if t == _K - 1:
            break
        d2 = jnp.sum((feats_ref[...] - center) ** 2, axis=1, keepdims=True)
        dist = jnp.sqrt(jnp.maximum(d2, 0.0))
        min_d = jnp.minimum(min_d, dist)
        probs = min_d * min_d
        s = jnp.sum(probs)
        logits = jnp.log(probs / (s + 1e-8) + 1e-30)
        w = logits + eg_ref[0, :, t:t + 1]
        m = jnp.max(w)
        idx = jnp.min(jnp.where(w >= m, iota, n))


def _body(idx0_ref, x_ref, a8_ref, a4_ref, eg4_ref, eg8_ref, eg16_ref,
          out_ref, p4_ref, p8_ref, p16_ref):
    b = pl.program_id(0)
    x = x_ref[0]
    sh = x[:, 0] + x[:, 1]
    p16 = (sh[:, :, :_D] + sh[:, :, _D:]) * 0.25
    for i in range(16):
        p16_ref[pl.ds(i * 16, 16), :] = p16[i]
    p16f = p16_ref[...]
    p8f = jnp.dot(a8_ref[...], p16f, preferred_element_type=jnp.float32)
    p8_ref[...] = p8f
    p4_ref[...] = jnp.dot(a4_ref[...], p8f, preferred_element_type=jnp.float32)

    scale_refs = (p4_ref, p8_ref, p16_ref)
    eg_refs = (eg4_ref, eg8_ref, eg16_ref)
    for si in range(3):
        idx0 = idx0_ref[b, si]
        _kmeans_pp_steps(scale_refs[si], _SCALE_NS[si], eg_refs[si], idx0,
                         out_ref, si * _K)


def kernel(features):
    b, h, w, d = features.shape
    idx0s, gumbels = _rng_consts()
    x = features.reshape(b, h // 2, 2, w // 2, 2 * d)

    grid_spec = pltpu.PrefetchScalarGridSpec(
        num_scalar_prefetch=1,
        grid=(b,),
        in_specs=[
            pl.BlockSpec((1, h // 2, 2, w // 2, 2 * d),
                         lambda i, s: (i, 0, 0, 0, 0)),
            pl.BlockSpec((64, 256), lambda i, s: (0, 0)),
            pl.BlockSpec((16, 64), lambda i, s: (0, 0)),
            pl.BlockSpec((1, _SCALE_NS[0], _K - 1), lambda i, s: (i, 0, 0)),
            pl.BlockSpec((1, _SCALE_NS[1], _K - 1), lambda i, s: (i, 0, 0)),
            pl.BlockSpec((1, _SCALE_NS[2], _K - 1), lambda i, s: (i, 0, 0)),
        ],
        out_specs=pl.BlockSpec((1, 3 * _K, d), lambda i, s: (i, 0, 0)),
        scratch_shapes=[
            pltpu.VMEM((_SCALE_NS[0], d), jnp.float32),
            pltpu.VMEM((_SCALE_NS[1], d), jnp.float32),
            pltpu.VMEM((_SCALE_NS[2], d), jnp.float32),
        ],
    )
    return pl.pallas_call(
        _body,
        grid_spec=grid_spec,
        out_shape=jax.ShapeDtypeStruct((b, 3 * _K, d), jnp.float32),
    )(jnp.asarray(idx0s), x, jnp.asarray(_pool_matrix(8, 16)),
      jnp.asarray(_pool_matrix(4, 8)), gumbels[0], gumbels[1], gumbels[2])


# speedup vs baseline: 2.6774x; 2.6774x over previous
import functools

import jax
import jax.numpy as jnp
import numpy as np
from jax.experimental import pallas as pl
from jax.experimental.pallas import tpu as pltpu

_B = 64
_D = 192
_SCALE_NS = (16, 64, 256)
_K = 4
_C = 8


@functools.lru_cache(maxsize=1)
def _rng_consts():
    with jax.ensure_compile_time_eval():
        base = jax.random.key(42)
        onehots = []
        gumbels = []
        for si, n in enumerate(_SCALE_NS):
            keys = jax.random.split(jax.random.fold_in(base, si), _B)

            def per_key(kk, n=n):
                key, sub = jax.random.split(kk)
                i0 = jax.random.randint(sub, (), 0, n)
                gs = []
                for _ in range(_K - 1):
                    key, sub = jax.random.split(key)
                    gs.append(jax.random.gumbel(sub, (n,), jnp.float32))
                return i0, jnp.stack(gs, axis=0)

            i0, g = jax.vmap(per_key)(keys)
            i0 = np.asarray(i0, np.int64)
            oh = np.zeros((_B, n), np.float32)
            oh[np.arange(_B), i0] = 1.0
            onehots.append(oh)
            gumbels.append(np.asarray(g, np.float32).transpose(1, 0, 2))
        return onehots, gumbels


def _pool_matrix(g_out, g_in):
    m = np.zeros((g_out * g_out, g_in * g_in), np.float32)
    for n in range(g_in * g_in):
        a, bcol = n // g_in, n % g_in
        m[(a // 2) * g_out + (bcol // 2), n] = 0.25
    return m


def _kmeans_pp(feats, n, oh0, eg, out_ref, out_base):
    centers = jnp.sum(oh0[:, :, None] * feats, axis=1)
    min_d = jnp.full((_C, n), jnp.float32(np.inf), jnp.float32)
    iota = jax.lax.broadcasted_iota(jnp.int32, (_C, n), 1)
    for t in range(_K):
        out_ref[:, out_base + t, :] = centers
        if t == _K - 1:
            break
        d2 = jnp.sum((feats - centers[:, None, :]) ** 2, axis=2)
        dist = jnp.sqrt(jnp.maximum(d2, 0.0))
        min_d = jnp.minimum(min_d, dist)
        probs = min_d * min_d
        s = jnp.sum(probs, axis=1, keepdims=True)
        logits = jnp.log(probs / (s + 1e-8) + 1e-30)
        w = logits + eg[t]
        m = jnp.max(w, axis=1, keepdims=True)
        idx = jnp.min(jnp.where(w >= m, iota, n), axis=1, keepdims=True)
        onehot = (iota == idx).astype(jnp.float32)
        centers = jnp.sum(onehot[:, :, None] * feats, axis=1)


def _body(x_ref, a8_ref, a4_ref, oh4_ref, oh8_ref, oh16_ref,
          eg4_ref, eg8_ref, eg16_ref, out_ref, p4_ref, p8_ref, p16_ref):
    x = x_ref[...]
    sh = x[:, :, 0] + x[:, :, 1]
    p16 = (sh[:, :, :, :_D] + sh[:, :, :, _D:]) * 0.25
    for i in range(16):
        p16_ref[:, pl.ds(i * 16, 16), :] = p16[:, i]
    for j in range(_C):
        p8j = jnp.dot(a8_ref[...], p16_ref[j], preferred_element_type=jnp.float32)
        p8_ref[j, :, :] = p8j
        p4_ref[j, :, :] = jnp.dot(a4_ref[...], p8j,
                                  preferred_element_type=jnp.float32)

    scale_refs = (p4_ref, p8_ref, p16_ref)
    oh_refs = (oh4_ref, oh8_ref, oh16_ref)
    eg_refs = (eg4_ref, eg8_ref, eg16_ref)
    for si in range(3):
        _kmeans_pp(scale_refs[si][...], _SCALE_NS[si], oh_refs[si][...],
                   eg_refs[si][...], out_ref, si * _K)


def kernel(features):
    b, h, w, d = features.shape
    onehots, gumbels = _rng_consts()
    x = features.reshape(b, h // 2, 2, w // 2, 2 * d)

    n4, n8, n16 = _SCALE_NS
    grid_spec = pltpu.PrefetchScalarGridSpec(
        num_scalar_prefetch=0,
        grid=(b // _C,),
        in_specs=[
            pl.BlockSpec((_C, h // 2, 2, w // 2, 2 * d),
                         lambda i: (i, 0, 0, 0, 0)),
            pl.BlockSpec((64, 256), lambda i: (0, 0)),
            pl.BlockSpec((16, 64), lambda i: (0, 0)),
            pl.BlockSpec((_C, n4), lambda i: (i, 0)),
            pl.BlockSpec((_C, n8), lambda i: (i, 0)),
            pl.BlockSpec((_C, n16), lambda i: (i, 0)),
            pl.BlockSpec((_K - 1, _C, n4), lambda i: (0, i, 0)),
            pl.BlockSpec((_K - 1, _C, n8), lambda i: (0, i, 0)),
            pl.BlockSpec((_K - 1, _C, n16), lambda i: (0, i, 0)),
        ],
        out_specs=pl.BlockSpec((_C, 3 * _K, d), lambda i: (i, 0, 0)),
        scratch_shapes=[
            pltpu.VMEM((_C, n4, d), jnp.float32),
            pltpu.VMEM((_C, n8, d), jnp.float32),
            pltpu.VMEM((_C, n16, d), jnp.float32),
        ],
    )
    return pl.pallas_call(
        _body,
        grid_spec=grid_spec,
        out_shape=jax.ShapeDtypeStruct((b, 3 * _K, d), jnp.float32),
    )(x, jnp.asarray(_pool_matrix(8, 16)), jnp.asarray(_pool_matrix(4, 8)),
      jnp.asarray(onehots[0]), jnp.asarray(onehots[1]), jnp.asarray(onehots[2]),
      jnp.asarray(gumbels[0]), jnp.asarray(gumbels[1]), jnp.asarray(gumbels[2]))


# speedup vs baseline: 3.0959x; 1.1563x over previous
import functools

import jax
import jax.numpy as jnp
import numpy as np
from jax.experimental import pallas as pl
from jax.experimental.pallas import tpu as pltpu

_B = 64
_D = 192
_SCALE_NS = (16, 64, 256)
_K = 4
_C = 16


@functools.lru_cache(maxsize=1)
def _rng_consts():
    with jax.ensure_compile_time_eval():
        base = jax.random.key(42)
        onehots = []
        expgs = []
        for si, n in enumerate(_SCALE_NS):
            keys = jax.random.split(jax.random.fold_in(base, si), _B)

            def per_key(kk, n=n):
                key, sub = jax.random.split(kk)
                i0 = jax.random.randint(sub, (), 0, n)
                gs = []
                for _ in range(_K - 1):
                    key, sub = jax.random.split(key)
                    gs.append(jax.random.gumbel(sub, (n,), jnp.float32))
                return i0, jnp.exp(jnp.stack(gs, axis=0))

            i0, g = jax.vmap(per_key)(keys)
            i0 = np.asarray(i0, np.int64)
            oh = np.zeros((_B, n), np.float32)
            oh[np.arange(_B), i0] = 1.0
            onehots.append(oh)
            expgs.append(np.asarray(g, np.float32).transpose(1, 0, 2))
        return onehots, expgs


def _pool_matrix(g_out, g_in):
    m = np.zeros((g_out * g_out, g_in * g_in), np.float32)
    for n in range(g_in * g_in):
        a, bcol = n // g_in, n % g_in
        m[(a // 2) * g_out + (bcol // 2), n] = 0.25
    return m


def _body(x_ref, a8_ref, a4_ref, oh4_ref, oh8_ref, oh16_ref,
          eg4_ref, eg8_ref, eg16_ref, out_ref,
          p4_ref, p8_ref, p16_ref, g4_ref, g8_ref, g16_ref, d4_ref, d8_ref, d16_ref):
    x = x_ref[...]
    sh = x[:, :, 0] + x[:, :, 1]
    p16 = (sh[:, :, :, :_D] + sh[:, :, :, _D:]) * 0.25
    for i in range(16):
        p16_ref[:, pl.ds(i * 16, 16), :] = p16[:, i]
    for j in range(_C):
        p8j = jnp.dot(a8_ref[...], p16_ref[j], preferred_element_type=jnp.float32)
        p8_ref[j, :, :] = p8j
        p4_ref[j, :, :] = jnp.dot(a4_ref[...], p8j,
                                  preferred_element_type=jnp.float32)

    scale_refs = (p4_ref, p8_ref, p16_ref)
    gram_refs = (g4_ref, g8_ref, g16_ref)
    dot_refs = (d4_ref, d8_ref, d16_ref)
    oh_refs = (oh4_ref, oh8_ref, oh16_ref)
    eg_refs = (eg4_ref, eg8_ref, eg16_ref)
    dn_t = (((1,), (1,)), ((), ()))

    sqs = [None] * 3
    ohs = [None] * 3
    min_d2 = [None] * 3
    iotas = [None] * 3
    for si in range(3):
        n = _SCALE_NS[si]
        feats = scale_refs[si][...]
        sqs[si] = jnp.sum(feats * feats, axis=2)
        for j in range(_C):
            gram_refs[si][j, :, :] = jax.lax.dot_general(
                feats[j], feats[j], dn_t, preferred_element_type=jnp.float32)
        ohs[si] = oh_refs[si][...]
        min_d2[si] = jnp.full((_C, n), jnp.float32(np.inf), jnp.float32)
        iotas[si] = jax.lax.broadcasted_iota(jnp.int32, (_C, n), 1)

    def emit_centers(si, t, oh):
        for j in range(_C):
            out_ref[j, pl.ds(si * _K + t, 1), :] = jnp.dot(
                oh[j:j + 1, :], scale_refs[si][j],
                preferred_element_type=jnp.float32)

    for t in range(_K - 1):
        for si in range(3):
            n = _SCALE_NS[si]
            oh = ohs[si]
            emit_centers(si, t, oh)
            dref = dot_refs[si]
            for j in range(_C):
                dref[pl.ds(j, 1), :] = jnp.dot(
                    oh[j:j + 1, :], gram_refs[si][j],
                    preferred_element_type=jnp.float32)
            dotrow = dref[...]
            cc = jnp.sum(oh * sqs[si], axis=1, keepdims=True)
            d2 = jnp.maximum(sqs[si] + cc - 2.0 * dotrow, 0.0)
            min_d2[si] = jnp.minimum(min_d2[si], d2)
            w = (min_d2[si] + 1e-38) * eg_refs[si][t]
            m = jnp.max(w, axis=1, keepdims=True)
            idx = jnp.min(jnp.where(w >= m, iotas[si], n), axis=1,
                          keepdims=True)
            ohs[si] = (iotas[si] == idx).astype(jnp.float32)

    for si in range(3):
        emit_centers(si, _K - 1, ohs[si])


def kernel(features):
    b, h, w, d = features.shape
    onehots, expgs = _rng_consts()
    x = features.reshape(b, h // 2, 2, w // 2, 2 * d)

    n4, n8, n16 = _SCALE_NS
    grid_spec = pltpu.PrefetchScalarGridSpec(
        num_scalar_prefetch=0,
        grid=(b // _C,),
        in_specs=[
            pl.BlockSpec((_C, h // 2, 2, w // 2, 2 * d),
                         lambda i: (i, 0, 0, 0, 0)),
            pl.BlockSpec((64, 256), lambda i: (0, 0)),
            pl.BlockSpec((16, 64), lambda i: (0, 0)),
            pl.BlockSpec((_C, n4), lambda i: (i, 0)),
            pl.BlockSpec((_C, n8), lambda i: (i, 0)),
            pl.BlockSpec((_C, n16), lambda i: (i, 0)),
            pl.BlockSpec((_K - 1, _C, n4), lambda i: (0, i, 0)),
            pl.BlockSpec((_K - 1, _C, n8), lambda i: (0, i, 0)),
            pl.BlockSpec((_K - 1, _C, n16), lambda i: (0, i, 0)),
        ],
        out_specs=pl.BlockSpec((_C, 3 * _K, d), lambda i: (i, 0, 0)),
        scratch_shapes=[
            pltpu.VMEM((_C, n4, d), jnp.float32),
            pltpu.VMEM((_C, n8, d), jnp.float32),
            pltpu.VMEM((_C, n16, d), jnp.float32),
            pltpu.VMEM((_C, n4, n4), jnp.float32),
            pltpu.VMEM((_C, n8, n8), jnp.float32),
            pltpu.VMEM((_C, n16, n16), jnp.float32),
            pltpu.VMEM((_C, n4), jnp.float32),
            pltpu.VMEM((_C, n8), jnp.float32),
            pltpu.VMEM((_C, n16), jnp.float32),
        ],
    )
    return pl.pallas_call(
        _body,
        grid_spec=grid_spec,
        out_shape=jax.ShapeDtypeStruct((b, 3 * _K, d), jnp.float32),
    )(x, jnp.asarray(_pool_matrix(8, 16)), jnp.asarray(_pool_matrix(4, 8)),
      jnp.asarray(onehots[0]), jnp.asarray(onehots[1]), jnp.asarray(onehots[2]),
      jnp.asarray(expgs[0]), jnp.asarray(expgs[1]), jnp.asarray(expgs[2]))
